# Initial kernel scaffold; baseline (speedup 1.0000x reference)
import functools

import jax
import jax.numpy as jnp
from jax import lax
from jax.experimental import pallas as pl
from jax.experimental.pallas import tpu as pltpu
from jax.experimental.pallas import tpu_sc as plsc

EMBED_DIM = 32
LANES = 128
NW = 32


def _sc_gather(x1g, x2g, T1, T2):
  G = x1g.shape[0]
  gpw = G // NW
  mesh = plsc.VectorSubcoreMesh(core_axis_name="c", subcore_axis_name="s")

  @functools.partial(
      pl.kernel,
      out_type=(
          jax.ShapeDtypeStruct((G, LANES, EMBED_DIM), jnp.float32),
          jax.ShapeDtypeStruct((G, LANES, EMBED_DIM), jnp.float32),
      ),
      mesh=mesh,
      scratch_types=(
          pltpu.VMEM((gpw, LANES), jnp.int32),
          pltpu.VMEM((gpw, LANES), jnp.int32),
          pltpu.VMEM((LANES, EMBED_DIM), jnp.float32),
          pltpu.VMEM((LANES, EMBED_DIM), jnp.float32),
          pltpu.SemaphoreType.DMA,
          pltpu.SemaphoreType.DMA,
      ),
  )
  def gather_kernel(x1_hbm, x2_hbm, t1_hbm, t2_hbm, e1_hbm, e2_hbm,
                    idx1_v, idx2_v, rows1_v, rows2_v, sem1, sem2):
    wid = lax.axis_index("s") * 2 + lax.axis_index("c")
    base = wid * gpw
    pltpu.sync_copy(x1_hbm.at[pl.ds(base, gpw)], idx1_v)
    pltpu.sync_copy(x2_hbm.at[pl.ds(base, gpw)], idx2_v)

    def body(g, carry):
      cp1 = pltpu.async_copy(t1_hbm.at[idx1_v.at[g]], rows1_v, sem1)
      cp2 = pltpu.async_copy(t2_hbm.at[idx2_v.at[g]], rows2_v, sem2)
      cp1.wait()
      cp2.wait()
      pltpu.sync_copy(rows1_v, e1_hbm.at[base + g])
      pltpu.sync_copy(rows2_v, e2_hbm.at[base + g])
      return carry

    lax.fori_loop(0, gpw, body, 0)

  return gather_kernel(x1g, x2g, T1, T2)


def _tc_mlp(e1, e2, w0a, w0b, b0, w1, b1, w2, b2, wpt, bp):
  n = e1.shape[0]
  blk = 8192
  grid = n // blk
  d = w1.shape[0]

  def mlp_kernel(e1_ref, e2_ref, w0a_ref, w0b_ref, b0_ref, w1_ref, b1_ref,
                 w2_ref, b2_ref, wpt_ref, bp_ref, out_ref):
    x = jnp.dot(e1_ref[...], w0a_ref[...], preferred_element_type=jnp.float32)
    x = x + jnp.dot(e2_ref[...], w0b_ref[...], preferred_element_type=jnp.float32)
    x = jnp.maximum(x + b0_ref[...], 0.0)
    x = jnp.maximum(
        jnp.dot(x, w1_ref[...], preferred_element_type=jnp.float32) + b1_ref[...], 0.0)
    x = jnp.maximum(
        jnp.dot(x, w2_ref[...], preferred_element_type=jnp.float32) + b2_ref[...], 0.0)
    z = jnp.sum(x * wpt_ref[...], axis=1) + bp_ref[0, 0]
    out_ref[...] = jax.nn.sigmoid(z)

  full = lambda shape: pl.BlockSpec(shape, lambda i: (0,) * len(shape))
  return pl.pallas_call(
      mlp_kernel,
      grid=(grid,),
      in_specs=[
          pl.BlockSpec((blk, EMBED_DIM), lambda i: (i, 0)),
          pl.BlockSpec((blk, EMBED_DIM), lambda i: (i, 0)),
          full((EMBED_DIM, d)),
          full((EMBED_DIM, d)),
          full((1, d)),
          full((d, d)),
          full((1, d)),
          full((d, d)),
          full((1, d)),
          full((1, d)),
          full((1, 1)),
      ],
      out_specs=pl.BlockSpec((blk,), lambda i: (i,)),
      out_shape=jax.ShapeDtypeStruct((n,), jnp.float32),
  )(e1, e2, w0a, w0b, b0, w1, b1, w2, b2, wpt, bp)


def kernel(x1, x2, T1, T2, W0, b0, W1, b1, W2, b2, Wp, bp):
  B, L = x1.shape
  n = B * L
  x1g = x1.astype(jnp.int32).reshape(n // LANES, LANES)
  x2g = x2.astype(jnp.int32).reshape(n // LANES, LANES)
  e1, e2 = _sc_gather(x1g, x2g, T1, T2)
  e1 = e1.reshape(n, EMBED_DIM)
  e2 = e2.reshape(n, EMBED_DIM)
  out = _tc_mlp(
      e1, e2,
      W0[:EMBED_DIM], W0[EMBED_DIM:], b0.reshape(1, -1),
      W1, b1.reshape(1, -1), W2, b2.reshape(1, -1),
      Wp.reshape(1, -1), bp.reshape(1, 1))
  return out.reshape(B, L)


# speedup vs baseline: 6.8426x; 6.8426x over previous
import functools

import jax
import jax.numpy as jnp
from jax import lax
from jax.experimental import pallas as pl
from jax.experimental.pallas import tpu as pltpu
from jax.experimental.pallas import tpu_sc as plsc

EMBED_DIM = 32
LANES = 128
NW = 32


def _sc_gather(x1g, x2g, T1, T2):
  G = x1g.shape[0] * x1g.shape[1]
  gpw = x1g.shape[1]
  mesh = plsc.VectorSubcoreMesh(core_axis_name="c", subcore_axis_name="s")

  @functools.partial(
      pl.kernel,
      out_type=(
          jax.ShapeDtypeStruct((G, LANES, EMBED_DIM), jnp.float32),
          jax.ShapeDtypeStruct((G, LANES, EMBED_DIM), jnp.float32),
      ),
      mesh=mesh,
      compiler_params=pltpu.CompilerParams(use_tc_tiling_on_sc=False),
      scratch_types=(
          pltpu.VMEM((gpw, LANES), jnp.int32),
          pltpu.VMEM((gpw, LANES), jnp.int32),
          pltpu.VMEM((LANES, EMBED_DIM), jnp.float32),
          pltpu.VMEM((LANES, EMBED_DIM), jnp.float32),
          pltpu.SemaphoreType.DMA,
          pltpu.SemaphoreType.DMA,
      ),
  )
  def gather_kernel(x1_hbm, x2_hbm, t1_hbm, t2_hbm, e1_hbm, e2_hbm,
                    idx1_v, idx2_v, rows1_v, rows2_v, sem1, sem2):
    wid = lax.axis_index("s") * 2 + lax.axis_index("c")
    base = wid * gpw
    pltpu.sync_copy(x1_hbm.at[wid], idx1_v)
    pltpu.sync_copy(x2_hbm.at[wid], idx2_v)

    def body(g, carry):
      cp1 = pltpu.async_copy(t1_hbm.at[idx1_v.at[g]], rows1_v, sem1)
      cp2 = pltpu.async_copy(t2_hbm.at[idx2_v.at[g]], rows2_v, sem2)
      cp1.wait()
      cp2.wait()
      pltpu.sync_copy(rows1_v, e1_hbm.at[base + g])
      pltpu.sync_copy(rows2_v, e2_hbm.at[base + g])
      return carry

    lax.fori_loop(0, gpw, body, 0)

  return gather_kernel(x1g, x2g, T1, T2)


def _tc_mlp(e1, e2, w0a, w0b, b0, w1, b1, w2, b2, wpt, bp):
  n = e1.shape[0]
  blk = 8192
  grid = n // blk
  d = w1.shape[0]

  def mlp_kernel(e1_ref, e2_ref, w0a_ref, w0b_ref, b0_ref, w1_ref, b1_ref,
                 w2_ref, b2_ref, wpt_ref, bp_ref, out_ref):
    x = jnp.dot(e1_ref[...], w0a_ref[...], preferred_element_type=jnp.float32)
    x = x + jnp.dot(e2_ref[...], w0b_ref[...], preferred_element_type=jnp.float32)
    x = jnp.maximum(x + b0_ref[...], 0.0)
    x = jnp.maximum(
        jnp.dot(x, w1_ref[...], preferred_element_type=jnp.float32) + b1_ref[...], 0.0)
    x = jnp.maximum(
        jnp.dot(x, w2_ref[...], preferred_element_type=jnp.float32) + b2_ref[...], 0.0)
    z = jnp.sum(x * wpt_ref[...], axis=1) + bp_ref[0, 0]
    out_ref[...] = jax.nn.sigmoid(z)

  full = lambda shape: pl.BlockSpec(shape, lambda i: (0,) * len(shape))
  return pl.pallas_call(
      mlp_kernel,
      grid=(grid,),
      in_specs=[
          pl.BlockSpec((blk, EMBED_DIM), lambda i: (i, 0)),
          pl.BlockSpec((blk, EMBED_DIM), lambda i: (i, 0)),
          full((EMBED_DIM, d)),
          full((EMBED_DIM, d)),
          full((1, d)),
          full((d, d)),
          full((1, d)),
          full((d, d)),
          full((1, d)),
          full((1, d)),
          full((1, 1)),
      ],
      out_specs=pl.BlockSpec((blk,), lambda i: (i,)),
      out_shape=jax.ShapeDtypeStruct((n,), jnp.float32),
  )(e1, e2, w0a, w0b, b0, w1, b1, w2, b2, wpt, bp)


def kernel(x1, x2, T1, T2, W0, b0, W1, b1, W2, b2, Wp, bp):
  B, L = x1.shape
  n = B * L
  x1g = x1.astype(jnp.int32).reshape(NW, n // (NW * LANES), LANES)
  x2g = x2.astype(jnp.int32).reshape(NW, n // (NW * LANES), LANES)
  e1, e2 = _sc_gather(x1g, x2g, T1, T2)
  e1 = e1.reshape(n, EMBED_DIM)
  e2 = e2.reshape(n, EMBED_DIM)
  out = _tc_mlp(
      e1, e2,
      W0[:EMBED_DIM], W0[EMBED_DIM:], b0.reshape(1, -1),
      W1, b1.reshape(1, -1), W2, b2.reshape(1, -1),
      Wp.reshape(1, -1), bp.reshape(1, 1))
  return out.reshape(B, L)


# speedup vs baseline: 8.0316x; 1.1738x over previous
import functools

import jax
import jax.numpy as jnp
from jax import lax
from jax.experimental import pallas as pl
from jax.experimental.pallas import tpu as pltpu
from jax.experimental.pallas import tpu_sc as plsc

EMBED_DIM = 32
LANES = 128
NW = 32


def _sc_gather(x1g, x2g, T1, T2):
  gpw = x1g.shape[1]
  n = NW * gpw * LANES
  mesh = plsc.VectorSubcoreMesh(core_axis_name="c", subcore_axis_name="s")

  @functools.partial(
      pl.kernel,
      out_type=jax.ShapeDtypeStruct((n, LANES), jnp.float32),
      mesh=mesh,
      compiler_params=pltpu.CompilerParams(use_tc_tiling_on_sc=False),
      scratch_types=(
          pltpu.VMEM((gpw, LANES), jnp.int32),
          pltpu.VMEM((gpw, LANES), jnp.int32),
          pltpu.VMEM((LANES, EMBED_DIM), jnp.float32),
          pltpu.VMEM((LANES, EMBED_DIM), jnp.float32),
          pltpu.SemaphoreType.DMA,
          pltpu.SemaphoreType.DMA,
      ),
  )
  def gather_kernel(x1_hbm, x2_hbm, t1_hbm, t2_hbm, ecat_hbm,
                    idx1_v, idx2_v, rows1_v, rows2_v, sem1, sem2):
    wid = lax.axis_index("s") * 2 + lax.axis_index("c")
    base = wid * gpw
    pltpu.sync_copy(x1_hbm.at[wid], idx1_v)
    pltpu.sync_copy(x2_hbm.at[wid], idx2_v)

    def body(g, carry):
      cp1 = pltpu.async_copy(t1_hbm.at[idx1_v.at[g]], rows1_v, sem1)
      cp2 = pltpu.async_copy(t2_hbm.at[idx2_v.at[g]], rows2_v, sem2)
      cp1.wait()
      cp2.wait()
      tok0 = (base + g) * LANES
      pltpu.sync_copy(rows1_v, ecat_hbm.at[pl.ds(tok0, LANES), pl.ds(0, EMBED_DIM)])
      pltpu.sync_copy(rows2_v,
                      ecat_hbm.at[pl.ds(tok0, LANES), pl.ds(EMBED_DIM, EMBED_DIM)])
      return carry

    lax.fori_loop(0, gpw, body, 0)

  return gather_kernel(x1g, x2g, T1, T2)


def _tc_mlp(ecat, w0, b0, w1, b1, w2, b2, wpt, bp):
  n = ecat.shape[0]
  blk = 8192
  grid = n // blk
  d = w1.shape[0]

  def mlp_kernel(ecat_ref, w0_ref, b0_ref, w1_ref, b1_ref,
                 w2_ref, b2_ref, wpt_ref, bp_ref, out_ref):
    e = ecat_ref[:, :d]
    x = jnp.dot(e, w0_ref[...], preferred_element_type=jnp.float32)
    x = jnp.maximum(x + b0_ref[...], 0.0)
    x = jnp.maximum(
        jnp.dot(x, w1_ref[...], preferred_element_type=jnp.float32) + b1_ref[...], 0.0)
    x = jnp.maximum(
        jnp.dot(x, w2_ref[...], preferred_element_type=jnp.float32) + b2_ref[...], 0.0)
    z = jnp.sum(x * wpt_ref[...], axis=1) + bp_ref[0, 0]
    out_ref[...] = jax.nn.sigmoid(z)

  full = lambda shape: pl.BlockSpec(shape, lambda i: (0,) * len(shape))
  return pl.pallas_call(
      mlp_kernel,
      grid=(grid,),
      in_specs=[
          pl.BlockSpec((blk, LANES), lambda i: (i, 0)),
          full((d, d)),
          full((1, d)),
          full((d, d)),
          full((1, d)),
          full((d, d)),
          full((1, d)),
          full((1, d)),
          full((1, 1)),
      ],
      out_specs=pl.BlockSpec((blk,), lambda i: (i,)),
      out_shape=jax.ShapeDtypeStruct((n,), jnp.float32),
  )(ecat, w0, b0, w1, b1, w2, b2, wpt, bp)


def kernel(x1, x2, T1, T2, W0, b0, W1, b1, W2, b2, Wp, bp):
  B, L = x1.shape
  n = B * L
  x1g = x1.astype(jnp.int32).reshape(NW, n // (NW * LANES), LANES)
  x2g = x2.astype(jnp.int32).reshape(NW, n // (NW * LANES), LANES)
  ecat = _sc_gather(x1g, x2g, T1, T2)
  out = _tc_mlp(
      ecat,
      W0, b0.reshape(1, -1),
      W1, b1.reshape(1, -1), W2, b2.reshape(1, -1),
      Wp.reshape(1, -1), bp.reshape(1, 1))
  return out.reshape(B, L)
